# baseline (speedup 1.0000x reference)
import functools

import jax
import jax.numpy as jnp
from jax import lax
from jax.experimental import pallas as pl
from jax.experimental.pallas import tpu as pltpu
from jax.experimental.pallas import tpu_sc as plsc

NC = 2
NS = 16
L = 16
NW = NC * NS
B = 16384
F = 64
BPW = B // NW
CH = 128
NCH = BPW // CH
RC = 128
NRC = BPW // RC

_mesh = plsc.VectorSubcoreMesh(core_axis_name="c", subcore_axis_name="s")


@functools.partial(
    pl.kernel,
    out_type=jax.ShapeDtypeStruct((B,), jnp.float32),
    mesh=_mesh,
    compiler_params=pltpu.CompilerParams(needs_layout_passes=False),
    scratch_types=[
        pltpu.VMEM((BPW,), jnp.int32),
        pltpu.VMEM((BPW,), jnp.int32),
        pltpu.VMEM((RC, F), jnp.float32),
        pltpu.VMEM((RC, F), jnp.float32),
        pltpu.VMEM((BPW,), jnp.float32),
        pltpu.VMEM((BPW,), jnp.float32),
        pltpu.VMEM((BPW,), jnp.float32),
        pltpu.SemaphoreType.DMA,
        pltpu.SemaphoreType.DMA,
    ],
)
def _sc_predict(x1, x2, U, M, UB, MB, out,
                idx1, idx2, ur, mr, ubv, mbv, outv,
                sem, rsem):
    wid = lax.axis_index("s") * NC + lax.axis_index("c")
    base = wid * BPW

    pltpu.sync_copy(x1.at[pl.ds(base, BPW)], idx1)
    pltpu.sync_copy(x2.at[pl.ds(base, BPW)], idx2)

    copies = []
    for j in range(NCH):
        sl = pl.ds(j * CH, CH)
        copies.append(pltpu.async_copy(UB.at[idx1.at[sl]], ubv.at[sl], sem))
        copies.append(pltpu.async_copy(MB.at[idx2.at[sl]], mbv.at[sl], sem))

    lanes = lax.iota(jnp.int32, L)

    for c in range(NRC):
        cbase = c * RC

        @plsc.parallel_loop(0, RC // L)
        def fetch_group(g):
            uvec = idx1[pl.ds(cbase + g * L, L)]
            mvec = idx2[pl.ds(cbase + g * L, L)]
            for r in range(L):
                i = g * L + r
                pltpu.async_copy(U.at[pl.ds(uvec[r], 1)],
                                 ur.at[pl.ds(i, 1)], rsem)
                pltpu.async_copy(M.at[pl.ds(mvec[r], 1)],
                                 mr.at[pl.ds(i, 1)], rsem)

        def drain_row(i, carry):
            pltpu.make_async_copy(U.at[pl.ds(0, 1)], ur.at[pl.ds(0, 1)],
                                  rsem).wait()
            pltpu.make_async_copy(M.at[pl.ds(0, 1)], mr.at[pl.ds(0, 1)],
                                  rsem).wait()
            return carry

        lax.fori_loop(0, RC, drain_row, 0)

        @plsc.parallel_loop(0, RC // L)
        def group(g):
            p = jnp.zeros((L,), jnp.float32)
            for r in range(L):
                row = g * L + r
                acc = ur[row, pl.ds(0, L)] * mr[row, pl.ds(0, L)]
                for j in range(1, F // L):
                    acc = acc + (ur[row, pl.ds(j * L, L)]
                                 * mr[row, pl.ds(j * L, L)])
                p = jnp.where(lanes == r, jnp.sum(acc), p)
            outv[pl.ds(cbase + g * L, L)] = p

    for cp in copies:
        cp.wait()

    def bias_clip(g, carry):
        sl = pl.ds(g * L, L)
        p = outv[sl] + ubv[sl] + mbv[sl]
        outv[sl] = jnp.minimum(jnp.maximum(p, 0.0), 5.0)
        return carry

    lax.fori_loop(0, BPW // L, bias_clip, 0)
    pltpu.sync_copy(outv, out.at[pl.ds(base, BPW)])


def kernel(x1, x2, U, M, UB, MB):
    return _sc_predict(x1.astype(jnp.int32), x2.astype(jnp.int32),
                       U, M, UB.reshape(-1), MB.reshape(-1))


# speedup vs baseline: 1.4406x; 1.4406x over previous
import functools

import jax
import jax.numpy as jnp
from jax import lax
from jax.experimental import pallas as pl
from jax.experimental.pallas import tpu as pltpu
from jax.experimental.pallas import tpu_sc as plsc

NC = 2
NS = 16
L = 16
NW = NC * NS
B = 16384
F = 64
TR = 8
BPW = B // NW
CH = 128
NCH = BPW // CH
RC = 32
NRC = BPW // RC

_mesh = plsc.VectorSubcoreMesh(core_axis_name="c", subcore_axis_name="s")


@functools.partial(
    pl.kernel,
    out_type=jax.ShapeDtypeStruct((B,), jnp.float32),
    mesh=_mesh,
    compiler_params=pltpu.CompilerParams(needs_layout_passes=False),
    scratch_types=[
        pltpu.VMEM((BPW,), jnp.int32),
        pltpu.VMEM((BPW,), jnp.int32),
        pltpu.VMEM((RC, TR, F), jnp.float32),
        pltpu.VMEM((RC, TR, F), jnp.float32),
        pltpu.VMEM((BPW,), jnp.float32),
        pltpu.VMEM((BPW,), jnp.float32),
        pltpu.VMEM((BPW,), jnp.float32),
        pltpu.SemaphoreType.DMA,
        pltpu.SemaphoreType.DMA,
    ],
)
def _sc_predict(x1, x2, U3, M3, UB, MB, out,
                idx1, idx2, ub_, mb_, ubv, mbv, outv,
                sem, rsem):
    wid = lax.axis_index("s") * NC + lax.axis_index("c")
    base = wid * BPW

    pltpu.sync_copy(x1.at[pl.ds(base, BPW)], idx1)
    pltpu.sync_copy(x2.at[pl.ds(base, BPW)], idx2)

    copies = []
    for j in range(NCH):
        sl = pl.ds(j * CH, CH)
        copies.append(pltpu.async_copy(UB.at[idx1.at[sl]], ubv.at[sl], sem))
        copies.append(pltpu.async_copy(MB.at[idx2.at[sl]], mbv.at[sl], sem))

    lanes = lax.iota(jnp.int32, L)

    def chunk(c, carry):
        cbase = c * RC

        for g in range(RC // L):
            uvec = idx1[pl.ds(cbase + g * L, L)]
            mvec = idx2[pl.ds(cbase + g * L, L)]
            ut = uvec >> 3
            mt = mvec >> 3
            for r in range(L):
                k = g * L + r
                pltpu.async_copy(U3.at[pl.ds(ut[r], 1)],
                                 ub_.at[pl.ds(k, 1)], rsem)
                pltpu.async_copy(M3.at[pl.ds(mt[r], 1)],
                                 mb_.at[pl.ds(k, 1)], rsem)

        for k in range(RC):
            pltpu.make_async_copy(U3.at[pl.ds(0, 1)], ub_.at[pl.ds(0, 1)],
                                  rsem).wait()
            pltpu.make_async_copy(M3.at[pl.ds(0, 1)], mb_.at[pl.ds(0, 1)],
                                  rsem).wait()

        for g in range(RC // L):
            uvec = idx1[pl.ds(cbase + g * L, L)]
            mvec = idx2[pl.ds(cbase + g * L, L)]
            urow = uvec & 7
            mrow = mvec & 7
            p = jnp.zeros((L,), jnp.float32)
            for r in range(L):
                k = g * L + r
                ri = urow[r]
                rj = mrow[r]
                acc = ub_[k, ri, pl.ds(0, L)] * mb_[k, rj, pl.ds(0, L)]
                for j in range(1, F // L):
                    acc = acc + (ub_[k, ri, pl.ds(j * L, L)]
                                 * mb_[k, rj, pl.ds(j * L, L)])
                p = jnp.where(lanes == r, jnp.sum(acc), p)
            outv[pl.ds(cbase + g * L, L)] = p
        return carry

    lax.fori_loop(0, NRC, chunk, 0)

    for cp in copies:
        cp.wait()

    def bias_clip(g, carry):
        sl = pl.ds(g * L, L)
        p = outv[sl] + ubv[sl] + mbv[sl]
        outv[sl] = jnp.minimum(jnp.maximum(p, 0.0), 5.0)
        return carry

    lax.fori_loop(0, BPW // L, bias_clip, 0)
    pltpu.sync_copy(outv, out.at[pl.ds(base, BPW)])


def kernel(x1, x2, U, M, UB, MB):
    U3 = U.reshape(U.shape[0] // TR, TR, F)
    M3 = M.reshape(M.shape[0] // TR, TR, F)
    return _sc_predict(x1.astype(jnp.int32), x2.astype(jnp.int32),
                       U3, M3, UB.reshape(-1), MB.reshape(-1))


# speedup vs baseline: 1.6104x; 1.1179x over previous
import functools

import jax
import jax.numpy as jnp
from jax import lax
from jax.experimental import pallas as pl
from jax.experimental.pallas import tpu as pltpu
from jax.experimental.pallas import tpu_sc as plsc

NC = 2
NS = 16
L = 16
NW = NC * NS
B = 16384
F = 64
TR = 1
BPW = B // NW
CH = 128
NCH = BPW // CH
RC = 64
NRC = BPW // RC

_mesh = plsc.VectorSubcoreMesh(core_axis_name="c", subcore_axis_name="s")


@functools.partial(
    pl.kernel,
    out_type=jax.ShapeDtypeStruct((B,), jnp.float32),
    mesh=_mesh,
    compiler_params=pltpu.CompilerParams(needs_layout_passes=False),
    scratch_types=[
        pltpu.VMEM((BPW,), jnp.int32),
        pltpu.VMEM((BPW,), jnp.int32),
        pltpu.VMEM((RC, TR, F), jnp.float32),
        pltpu.VMEM((RC, TR, F), jnp.float32),
        pltpu.VMEM((BPW,), jnp.float32),
        pltpu.VMEM((BPW,), jnp.float32),
        pltpu.VMEM((BPW,), jnp.float32),
        pltpu.SemaphoreType.DMA,
        pltpu.SemaphoreType.DMA,
    ],
)
def _sc_predict(x1, x2, U3, M3, UB, MB, out,
                idx1, idx2, ub_, mb_, ubv, mbv, outv,
                sem, rsem):
    wid = lax.axis_index("s") * NC + lax.axis_index("c")
    base = wid * BPW

    pltpu.sync_copy(x1.at[pl.ds(base, BPW)], idx1)
    pltpu.sync_copy(x2.at[pl.ds(base, BPW)], idx2)

    copies = []
    for j in range(NCH):
        sl = pl.ds(j * CH, CH)
        copies.append(pltpu.async_copy(UB.at[idx1.at[sl]], ubv.at[sl], sem))
        copies.append(pltpu.async_copy(MB.at[idx2.at[sl]], mbv.at[sl], sem))

    lanes = lax.iota(jnp.int32, L)

    def chunk(c, carry):
        cbase = c * RC

        for g in range(RC // L):
            uvec = idx1[pl.ds(cbase + g * L, L)]
            mvec = idx2[pl.ds(cbase + g * L, L)]
            for r in range(L):
                k = g * L + r
                pltpu.async_copy(U3.at[pl.ds(uvec[r], 1)],
                                 ub_.at[pl.ds(k, 1)], rsem)
                pltpu.async_copy(M3.at[pl.ds(mvec[r], 1)],
                                 mb_.at[pl.ds(k, 1)], rsem)

        for k in range(RC):
            pltpu.make_async_copy(U3.at[pl.ds(0, 1)], ub_.at[pl.ds(0, 1)],
                                  rsem).wait()
            pltpu.make_async_copy(M3.at[pl.ds(0, 1)], mb_.at[pl.ds(0, 1)],
                                  rsem).wait()

        for g in range(RC // L):
            p = jnp.zeros((L,), jnp.float32)
            for r in range(L):
                k = g * L + r
                acc = ub_[k, 0, pl.ds(0, L)] * mb_[k, 0, pl.ds(0, L)]
                for j in range(1, F // L):
                    acc = acc + (ub_[k, 0, pl.ds(j * L, L)]
                                 * mb_[k, 0, pl.ds(j * L, L)])
                p = jnp.where(lanes == r, jnp.sum(acc), p)
            outv[pl.ds(cbase + g * L, L)] = p
        return carry

    lax.fori_loop(0, NRC, chunk, 0)

    for cp in copies:
        cp.wait()

    def bias_clip(g, carry):
        sl = pl.ds(g * L, L)
        p = outv[sl] + ubv[sl] + mbv[sl]
        outv[sl] = jnp.minimum(jnp.maximum(p, 0.0), 5.0)
        return carry

    lax.fori_loop(0, BPW // L, bias_clip, 0)
    pltpu.sync_copy(outv, out.at[pl.ds(base, BPW)])


def kernel(x1, x2, U, M, UB, MB):
    U3 = U.reshape(U.shape[0] // TR, TR, F)
    M3 = M.reshape(M.shape[0] // TR, TR, F)
    return _sc_predict(x1.astype(jnp.int32), x2.astype(jnp.int32),
                       U3, M3, UB.reshape(-1), MB.reshape(-1))


# speedup vs baseline: 1.6864x; 1.0472x over previous
import functools

import jax
import jax.numpy as jnp
from jax import lax
from jax.experimental import pallas as pl
from jax.experimental.pallas import tpu as pltpu
from jax.experimental.pallas import tpu_sc as plsc

NC = 2
NS = 16
L = 16
NW = NC * NS
B = 16384
F = 64
TR = 1
BPW = B // NW
CH = 128
NCH = BPW // CH
RC = 64
NRC = BPW // RC

_mesh = plsc.VectorSubcoreMesh(core_axis_name="c", subcore_axis_name="s")


@functools.partial(
    pl.kernel,
    out_type=jax.ShapeDtypeStruct((B,), jnp.float32),
    mesh=_mesh,
    compiler_params=pltpu.CompilerParams(needs_layout_passes=False),
    scratch_types=[
        pltpu.VMEM((BPW,), jnp.int32),
        pltpu.VMEM((BPW,), jnp.int32),
        pltpu.VMEM((RC, TR, F), jnp.float32),
        pltpu.VMEM((RC, TR, F), jnp.float32),
        pltpu.VMEM((BPW,), jnp.float32),
        pltpu.VMEM((BPW,), jnp.float32),
        pltpu.VMEM((BPW,), jnp.float32),
        pltpu.SemaphoreType.DMA,
        pltpu.SemaphoreType.DMA,
    ],
)
def _sc_predict(x1, x2, U3, M3, UB, MB, out,
                idx1, idx2, ub_, mb_, ubv, mbv, outv,
                sem, rsem):
    wid = lax.axis_index("s") * NC + lax.axis_index("c")
    base = wid * BPW

    pltpu.sync_copy(x1.at[pl.ds(base, BPW)], idx1)
    pltpu.sync_copy(x2.at[pl.ds(base, BPW)], idx2)

    copies = []
    for j in range(NCH):
        sl = pl.ds(j * CH, CH)
        copies.append(pltpu.async_copy(UB.at[idx1.at[sl]], ubv.at[sl], sem))
        copies.append(pltpu.async_copy(MB.at[idx2.at[sl]], mbv.at[sl], sem))

    lanes = lax.iota(jnp.int32, L)

    def chunk(c, carry):
        cbase = c * RC

        du = pltpu.async_copy(U3.at[idx1.at[pl.ds(cbase, RC)]], ub_, rsem)
        dm = pltpu.async_copy(M3.at[idx2.at[pl.ds(cbase, RC)]], mb_, rsem)
        du.wait()
        dm.wait()

        for g in range(RC // L):
            p = jnp.zeros((L,), jnp.float32)
            for r in range(L):
                k = g * L + r
                acc = ub_[k, 0, pl.ds(0, L)] * mb_[k, 0, pl.ds(0, L)]
                for j in range(1, F // L):
                    acc = acc + (ub_[k, 0, pl.ds(j * L, L)]
                                 * mb_[k, 0, pl.ds(j * L, L)])
                p = jnp.where(lanes == r, jnp.sum(acc), p)
            outv[pl.ds(cbase + g * L, L)] = p
        return carry

    lax.fori_loop(0, NRC, chunk, 0)

    for cp in copies:
        cp.wait()

    def bias_clip(g, carry):
        sl = pl.ds(g * L, L)
        p = outv[sl] + ubv[sl] + mbv[sl]
        outv[sl] = jnp.minimum(jnp.maximum(p, 0.0), 5.0)
        return carry

    lax.fori_loop(0, BPW // L, bias_clip, 0)
    pltpu.sync_copy(outv, out.at[pl.ds(base, BPW)])


def kernel(x1, x2, U, M, UB, MB):
    U3 = U.reshape(U.shape[0] // TR, TR, F)
    M3 = M.reshape(M.shape[0] // TR, TR, F)
    return _sc_predict(x1.astype(jnp.int32), x2.astype(jnp.int32),
                       U3, M3, UB.reshape(-1), MB.reshape(-1))
